# baseline (speedup 1.0000x reference)
import functools

import jax
import jax.numpy as jnp
from jax import lax
from jax.experimental import pallas as pl
from jax.experimental.pallas import tpu as pltpu
from jax.experimental.pallas import tpu_sc as plsc

NN = 10000
EE = 320000
FD = 128
NP = 10240
DUMMY = 10000
NTILES = 32
EPT = 10240
EP = EPT * NTILES
CH = 32
NCHUNK = EPT // CH
PADE = 128
KKS = (8000, 6400, 5120)
ROWS_B = 1024
GRID_R = NP // ROWS_B
NEGINF = float("-inf")

_HIGH = lax.Precision.HIGHEST


def _dot(a, b):
    return lax.dot_general(a, b, (((1,), (0,)), ((), ())),
                           preferred_element_type=jnp.float32,
                           precision=_HIGH)


def _dott(a, b):
    return lax.dot_general(a, b, (((0,), (0,)), ((), ())),
                           preferred_element_type=jnp.float32,
                           precision=_HIGH)


def _sortable_key(s):
    bu = lax.bitcast_convert_type(s, jnp.uint32)
    keyu = jnp.where((bu >> 31) != 0, ~bu, bu | jnp.uint32(0x80000000))
    return lax.bitcast_convert_type(keyu ^ jnp.uint32(0x80000000), jnp.int32)


def _mm_relu_body(x_ref, w_ref, b_ref, o_ref):
    o_ref[...] = jnp.maximum(_dot(x_ref[...], w_ref[...]) + b_ref[...], 0.0)


def _mm_relu(x, w, b2d):
    h = w.shape[1]
    return pl.pallas_call(
        _mm_relu_body,
        grid=(GRID_R,),
        in_specs=[pl.BlockSpec((ROWS_B, FD), lambda i: (i, 0)),
                  pl.BlockSpec((FD, h), lambda i: (0, 0)),
                  pl.BlockSpec((1, h), lambda i: (0, 0))],
        out_specs=pl.BlockSpec((ROWS_B, h), lambda i: (i, 0)),
        out_shape=jax.ShapeDtypeStruct((NP, h), jnp.float32),
    )(x, w, b2d)


def _deg_finish_body(degp_ref, h_ref, dinv_ref, curs_ref):
    ones = jnp.ones((NTILES, FD), jnp.float32)
    deg = _dott(degp_ref[...], ones) + 1.0
    dinv = lax.rsqrt(deg)
    dinv_ref[...] = dinv
    curs_ref[...] = dinv * h_ref[...]


def _deg_finish(degp, h):
    return pl.pallas_call(
        _deg_finish_body,
        grid=(GRID_R,),
        in_specs=[pl.BlockSpec((NTILES, ROWS_B), lambda i: (0, i)),
                  pl.BlockSpec((ROWS_B, FD), lambda i: (i, 0))],
        out_specs=[pl.BlockSpec((ROWS_B, FD), lambda i: (i, 0)),
                   pl.BlockSpec((ROWS_B, FD), lambda i: (i, 0))],
        out_shape=[jax.ShapeDtypeStruct((NP, FD), jnp.float32),
                   jax.ShapeDtypeStruct((NP, FD), jnp.float32)],
    )(degp, h)


def _combine1_body(part_ref, h0_ref, dinv_ref, g_ref, acc_ref, cur1_ref,
                   curs2_ref):
    agg = part_ref[0] + part_ref[1]
    dinv = dinv_ref[...]
    h0 = h0_ref[...]
    cur1 = dinv * agg + dinv * dinv * h0
    acc_ref[...] = g_ref[0] * h0 + g_ref[1] * cur1
    cur1_ref[...] = cur1
    curs2_ref[...] = dinv * cur1


def _combine1(part, h0, dinvb, g):
    return pl.pallas_call(
        _combine1_body,
        grid=(GRID_R,),
        in_specs=[pl.BlockSpec((2, ROWS_B, FD), lambda i: (0, i, 0)),
                  pl.BlockSpec((ROWS_B, FD), lambda i: (i, 0)),
                  pl.BlockSpec((ROWS_B, FD), lambda i: (i, 0)),
                  pl.BlockSpec(memory_space=pltpu.SMEM)],
        out_specs=[pl.BlockSpec((ROWS_B, FD), lambda i: (i, 0)),
                   pl.BlockSpec((ROWS_B, FD), lambda i: (i, 0)),
                   pl.BlockSpec((ROWS_B, FD), lambda i: (i, 0))],
        out_shape=[jax.ShapeDtypeStruct((NP, FD), jnp.float32)] * 3,
    )(part, h0, dinvb, g)


def _combine2_body(part_ref, cur1_ref, acc_ref, dinv_ref, g_ref, p_ref,
                   live_ref, hp_ref, sc_ref):
    agg = part_ref[0] + part_ref[1]
    dinv = dinv_ref[...]
    cur1 = cur1_ref[...]
    cur2 = dinv * agg + dinv * dinv * cur1
    hp = acc_ref[...] + g_ref[2] * cur2
    hp_ref[...] = hp
    p = p_ref[...]
    rn = lax.rsqrt(jnp.sum(p * p))
    s = _dot(hp, p) * rn
    sc_ref[...] = jnp.where(live_ref[...] > 0.0, s, NEGINF)


def _combine2(part, cur1, acc, dinvb, g, p2d, live2d):
    return pl.pallas_call(
        _combine2_body,
        grid=(GRID_R,),
        in_specs=[pl.BlockSpec((2, ROWS_B, FD), lambda i: (0, i, 0)),
                  pl.BlockSpec((ROWS_B, FD), lambda i: (i, 0)),
                  pl.BlockSpec((ROWS_B, FD), lambda i: (i, 0)),
                  pl.BlockSpec((ROWS_B, FD), lambda i: (i, 0)),
                  pl.BlockSpec(memory_space=pltpu.SMEM),
                  pl.BlockSpec((FD, 1), lambda i: (0, 0)),
                  pl.BlockSpec((ROWS_B, 1), lambda i: (i, 0))],
        out_specs=[pl.BlockSpec((ROWS_B, FD), lambda i: (i, 0)),
                   pl.BlockSpec((ROWS_B, 1), lambda i: (i, 0))],
        out_shape=[jax.ShapeDtypeStruct((NP, FD), jnp.float32),
                   jax.ShapeDtypeStruct((NP, 1), jnp.float32)],
    )(part, cur1, acc, dinvb, g, p2d, live2d)


def _topk_body(sc_ref, thr_ref, *, kk):
    key = _sortable_key(sc_ref[...])

    def vbody(i, t):
        cand = t + (jnp.int32(1) << (30 - i))
        cnt = jnp.sum((key >= cand).astype(jnp.int32))
        return jnp.where(cnt >= kk, cand, t)

    t0 = jnp.where(jnp.sum((key >= 0).astype(jnp.int32)) >= kk,
                   jnp.int32(0), jnp.int32(-2147483648))
    tval = lax.fori_loop(0, 31, vbody, t0)
    cgt = jnp.sum((key > tval).astype(jnp.int32))
    r = kk - cgt
    tie = key == tval
    ridx = (lax.broadcasted_iota(jnp.int32, key.shape, 0) * 128
            + lax.broadcasted_iota(jnp.int32, key.shape, 1))

    def ibody(i, c):
        cand = c + (jnp.int32(1) << (13 - i))
        cnt = jnp.sum((tie & (ridx < cand)).astype(jnp.int32))
        return jnp.where(cnt <= r, cand, c)

    cval = lax.fori_loop(0, 14, ibody, jnp.int32(0))
    thr_ref[...] = jnp.concatenate(
        [tval.reshape(1, 1), cval.reshape(1, 1)], axis=1)


def _topk(score2d, kk):
    return pl.pallas_call(
        functools.partial(_topk_body, kk=kk),
        in_specs=[pl.BlockSpec((NP // 128, 128), lambda: (0, 0))],
        out_specs=pl.BlockSpec((1, 2), lambda: (0, 0)),
        out_shape=jax.ShapeDtypeStruct((1, 2), jnp.int32),
    )(score2d)


def _pool_body(sc_ref, thr_ref, hp_ref, hn_ref, nl_ref, mx_ref, sm_ref, *,
               kk):
    i = pl.program_id(0)
    s = sc_ref[...]
    key = _sortable_key(s)
    tval = thr_ref[0, 0]
    cval = thr_ref[0, 1]
    ridx = lax.broadcasted_iota(jnp.int32, (ROWS_B, 1), 0) + i * ROWS_B
    keep = (key > tval) | ((key == tval) & (ridx < cval))
    keepf = keep.astype(jnp.float32)
    ones_r = jnp.ones((1, FD), jnp.float32)
    keepb = _dot(keepf, ones_r)
    th = jnp.tanh(jnp.clip(s, -30.0, 30.0))
    thb = _dot(th, ones_r)
    hn = jnp.where(keepb > 0.5, hp_ref[...] * thb, 0.0)
    hn_ref[...] = hn
    nl_ref[...] = keepf
    bmx = jnp.max(jnp.where(keepb > 0.5, hn, NEGINF), axis=0, keepdims=True)
    bsm = jnp.sum(hn, axis=0, keepdims=True) * (1.0 / kk)

    @pl.when(i == 0)
    def _():
        mx_ref[...] = jnp.full((1, FD), NEGINF, jnp.float32)
        sm_ref[...] = jnp.zeros((1, FD), jnp.float32)

    mx_ref[...] = jnp.maximum(mx_ref[...], bmx)
    sm_ref[...] = sm_ref[...] + bsm


def _pool(score, thr, hp, kk):
    return pl.pallas_call(
        functools.partial(_pool_body, kk=kk),
        grid=(GRID_R,),
        in_specs=[pl.BlockSpec((ROWS_B, 1), lambda i: (i, 0)),
                  pl.BlockSpec(memory_space=pltpu.SMEM),
                  pl.BlockSpec((ROWS_B, FD), lambda i: (i, 0))],
        out_specs=[pl.BlockSpec((ROWS_B, FD), lambda i: (i, 0)),
                   pl.BlockSpec((ROWS_B, 1), lambda i: (i, 0)),
                   pl.BlockSpec((1, FD), lambda i: (0, 0)),
                   pl.BlockSpec((1, FD), lambda i: (0, 0))],
        out_shape=[jax.ShapeDtypeStruct((NP, FD), jnp.float32),
                   jax.ShapeDtypeStruct((NP, 1), jnp.float32),
                   jax.ShapeDtypeStruct((1, FD), jnp.float32),
                   jax.ShapeDtypeStruct((1, FD), jnp.float32)],
    )(score, thr, hp)


def _head_body(xr_ref, w1_ref, b1_ref, w2_ref, b2_ref, w3_ref, b3_ref,
               o_ref):
    xs = xr_ref[0:1] + xr_ref[1:2] + xr_ref[2:3]
    o = jnp.maximum(_dot(xs, w1_ref[...]) + b1_ref[...], 0.0)
    o = jnp.maximum(_dot(o, w2_ref[...]) + b2_ref[...], 0.0)
    o = _dot(o, w3_ref[...]) + b3_ref[...]
    m = jnp.max(o, axis=1, keepdims=True)
    lse = jnp.log(jnp.sum(jnp.exp(o - m), axis=1, keepdims=True)) + m
    o_ref[...] = o - lse


def _head(xr, w1, b1, w2, b2, w3, b3):
    specs = [pl.BlockSpec(a.shape, lambda: (0,) * a.ndim)
             for a in (xr, w1, b1, w2, b2, w3, b3)]
    return pl.pallas_call(
        _head_body,
        in_specs=specs,
        out_specs=pl.BlockSpec((1, 40), lambda: (0, 0)),
        out_shape=jax.ShapeDtypeStruct((1, 40), jnp.float32),
    )(xr, w1, b1, w2, b2, w3, b3)


def _sc_edge_prep_body(src_hbm, dst_hbm, live_hbm, s2_hbm, d2_hbm, degp_hbm,
                       cnt_hbm, live_v, deg_v, src_v, dst_v, s2_v, d2_v,
                       cnt_v):
    c = lax.axis_index("c")
    s = lax.axis_index("s")
    wid = s * 2 + c
    pltpu.sync_copy(live_hbm, live_v)
    pltpu.sync_copy(src_hbm.at[wid], src_v)
    pltpu.sync_copy(dst_hbm.at[wid], dst_v)

    def zbody(i, carry):
        deg_v[pl.ds(i * 16, 16)] = jnp.zeros((16,), jnp.float32)
        return carry

    lax.fori_loop(0, NP // 16, zbody, 0)

    def vec(j, off):
        sl = pl.ds(j * 16, 16)
        sv = src_v[sl]
        dv = dst_v[sl]
        ew = plsc.load_gather(live_v, [sv]) * plsc.load_gather(live_v, [dv])
        keep = ew > 0.0
        plsc.addupdate_scatter(deg_v, [dv], ew)
        plsc.store_compressed(s2_v.at[pl.ds(off, 16)], sv, mask=keep)
        plsc.store_compressed(d2_v.at[pl.ds(off, 16)], dv, mask=keep)
        pop = plsc.all_reduce_population_count(keep)
        return off + jnp.max(pop)

    cnt = lax.fori_loop(0, EPT // 16, vec, jnp.int32(0))
    for j in range(PADE // 16):
        slj = pl.ds(cnt + j * 16, 16)
        s2_v[slj] = jnp.full((16,), DUMMY, jnp.int32)
        d2_v[slj] = jnp.full((16,), DUMMY, jnp.int32)
    cntp = ((cnt + PADE - 1) // PADE) * PADE
    cnt_v[...] = jnp.broadcast_to(cntp, (16,))
    pltpu.sync_copy(s2_v.at[pl.ds(0, EPT)], s2_hbm.at[wid])
    pltpu.sync_copy(d2_v.at[pl.ds(0, EPT)], d2_hbm.at[wid])
    pltpu.sync_copy(deg_v, degp_hbm.at[wid])
    pltpu.sync_copy(cnt_v, cnt_hbm.at[wid])


def _sc_hop_body(curs_hbm, s2_hbm, d2_hbm, cnt_hbm, z_hbm, part_hbm, acc,
                 idxs_v, idxd_v, rows0, rows1, rows2, rows3, cnt_v, sg0, sg1,
                 sg2, sg3, ss0, ss1, ss2, ss3):
    c = lax.axis_index("c")
    s = lax.axis_index("s")
    wid = s * 2 + c
    rpt = NP // 16
    zsl = pl.ds(s * rpt, rpt)
    pltpu.async_copy(z_hbm.at[zsl], acc.at[zsl], sg0)
    pltpu.async_copy(s2_hbm.at[wid], idxs_v, sg1)
    pltpu.async_copy(d2_hbm.at[wid], idxd_v, sg2)
    pltpu.async_copy(cnt_hbm.at[wid], cnt_v, sg3)
    pltpu.make_async_copy(z_hbm.at[zsl], acc.at[zsl], sg0).wait()
    pltpu.make_async_copy(s2_hbm.at[wid], idxs_v, sg1).wait()
    pltpu.make_async_copy(d2_hbm.at[wid], idxd_v, sg2).wait()
    pltpu.make_async_copy(cnt_hbm.at[wid], cnt_v, sg3).wait()
    plsc.subcore_barrier()

    def gstart(ci, rows, sem):
        pltpu.async_copy(curs_hbm.at[idxs_v.at[pl.ds(ci * CH, CH)]], rows,
                         sem)

    def gwait(ci, rows, sem):
        pltpu.make_async_copy(curs_hbm.at[idxs_v.at[pl.ds(ci * CH, CH)]],
                              rows, sem).wait()

    def sstart(ci, rows, sem):
        pltpu.async_copy(rows, acc.at[idxd_v.at[pl.ds(ci * CH, CH)]], sem,
                         add=True)

    def swait(ci, rows, sem):
        pltpu.make_async_copy(rows, acc.at[idxd_v.at[pl.ds(ci * CH, CH)]],
                              sem).wait()

    nq = cnt_v[...][0] // (4 * CH)

    @pl.when(nq > 0)
    def _():
        gstart(0, rows0, sg0)
        gstart(1, rows1, sg1)

    def quad(q, carry):
        c0 = 4 * q
        gwait(c0, rows0, sg0)
        sstart(c0, rows0, ss0)

        @pl.when(q > 0)
        def _():
            swait(c0 - 2, rows2, ss2)

        gstart(c0 + 2, rows2, sg2)
        gwait(c0 + 1, rows1, sg1)
        sstart(c0 + 1, rows1, ss1)

        @pl.when(q > 0)
        def _():
            swait(c0 - 1, rows3, ss3)

        gstart(c0 + 3, rows3, sg3)
        gwait(c0 + 2, rows2, sg2)
        sstart(c0 + 2, rows2, ss2)
        swait(c0, rows0, ss0)

        @pl.when(q < nq - 1)
        def _():
            gstart(c0 + 4, rows0, sg0)

        gwait(c0 + 3, rows3, sg3)
        sstart(c0 + 3, rows3, ss3)
        swait(c0 + 1, rows1, ss1)

        @pl.when(q < nq - 1)
        def _():
            gstart(c0 + 5, rows1, sg1)

        return carry

    lax.fori_loop(0, nq, quad, 0)

    @pl.when(nq > 0)
    def _():
        swait(4 * nq - 2, rows2, ss2)
        swait(4 * nq - 1, rows3, ss3)
    plsc.subcore_barrier()
    pltpu.sync_copy(acc.at[pl.ds(s * rpt, rpt)],
                    part_hbm.at[c, pl.ds(s * rpt, rpt)])


@functools.lru_cache(maxsize=None)
def _sc_kernels():
    mesh = plsc.VectorSubcoreMesh(core_axis_name="c", subcore_axis_name="s",
                                  num_cores=2, num_subcores=16)
    params = pltpu.CompilerParams(needs_layout_passes=False)
    edge_prep = pl.kernel(
        _sc_edge_prep_body,
        out_type=(jax.ShapeDtypeStruct((NTILES, EPT), jnp.int32),
                  jax.ShapeDtypeStruct((NTILES, EPT), jnp.int32),
                  jax.ShapeDtypeStruct((NTILES, NP), jnp.float32),
                  jax.ShapeDtypeStruct((NTILES, 16), jnp.int32)),
        mesh=mesh,
        scratch_types=[pltpu.VMEM((NP,), jnp.float32),
                       pltpu.VMEM((NP,), jnp.float32),
                       pltpu.VMEM((EPT,), jnp.int32),
                       pltpu.VMEM((EPT,), jnp.int32),
                       pltpu.VMEM((EPT + PADE,), jnp.int32),
                       pltpu.VMEM((EPT + PADE,), jnp.int32),
                       pltpu.VMEM((16,), jnp.int32)],
        compiler_params=params,
    )
    hop = pl.kernel(
        _sc_hop_body,
        out_type=jax.ShapeDtypeStruct((2, NP, FD), jnp.float32),
        mesh=mesh,
        scratch_types=[pltpu.VMEM_SHARED((NP, FD), jnp.float32),
                       pltpu.VMEM((EPT,), jnp.int32),
                       pltpu.VMEM((EPT,), jnp.int32),
                       pltpu.VMEM((CH, FD), jnp.float32),
                       pltpu.VMEM((CH, FD), jnp.float32),
                       pltpu.VMEM((CH, FD), jnp.float32),
                       pltpu.VMEM((CH, FD), jnp.float32),
                       pltpu.VMEM((16,), jnp.int32),
                       pltpu.SemaphoreType.DMA,
                       pltpu.SemaphoreType.DMA,
                       pltpu.SemaphoreType.DMA,
                       pltpu.SemaphoreType.DMA,
                       pltpu.SemaphoreType.DMA,
                       pltpu.SemaphoreType.DMA,
                       pltpu.SemaphoreType.DMA,
                       pltpu.SemaphoreType.DMA],
        compiler_params=params,
    )
    return edge_prep, hop


def _sc_edge_prep(srcp, dstp, live):
    return _sc_kernels()[0](srcp, dstp, live)


def _sc_hop(curs, s2, d2, cnts, zeros):
    return _sc_kernels()[1](curs, s2, d2, cnts, zeros)


def kernel(x, edge_index, batch, W12, b12, W22, b22, W32, b32, W1, b1, W2,
           b2, W3, b3, g1, g2, g3, p1, p2, p3):
    src = edge_index[0]
    dst = edge_index[1]
    pad_e = jnp.full((EP - EE,), DUMMY, jnp.int32)
    srcp = jnp.concatenate([src, pad_e]).reshape(NTILES, EPT)
    dstp = jnp.concatenate([dst, pad_e]).reshape(NTILES, EPT)
    xp = jnp.pad(x, ((0, NP - NN), (0, 0)))
    live = (jnp.arange(NP) < NN).astype(jnp.float32)

    h = _mm_relu(xp, W12, b12.reshape(1, -1))
    zeros = jnp.zeros((NP, FD), jnp.float32)
    gs = (g1, g2, g3)
    ps = (p1, p2, p3)
    nxt = ((W22, b22), (W32, b32), None)
    xrs = []
    for l in range(3):
        s2, d2, degp, cnts = _sc_edge_prep(srcp, dstp, live)
        dinvb, curs = _deg_finish(degp, h)
        part = _sc_hop(curs, s2, d2, cnts, zeros)
        accv, cur1, curs2 = _combine1(part, h, dinvb, gs[l])
        part2 = _sc_hop(curs2, s2, d2, cnts, zeros)
        hp, score = _combine2(part2, cur1, accv, dinvb, gs[l],
                              ps[l].reshape(-1, 1), live.reshape(-1, 1))
        thr = _topk(score.reshape(NP // 128, 128), KKS[l])
        hn, nl, mx, sm = _pool(score, thr, hp, KKS[l])
        xrs.append(jnp.concatenate([mx, sm], axis=1))
        live = nl.reshape(-1)
        if nxt[l] is not None:
            h = _mm_relu(hn, nxt[l][0], nxt[l][1].reshape(1, -1))
    return _head(jnp.concatenate(xrs, 0), W1, b1.reshape(1, -1),
                 W2, b2.reshape(1, -1), W3, b3.reshape(1, -1))


# speedup vs baseline: 1.0162x; 1.0162x over previous
import functools

import jax
import jax.numpy as jnp
from jax import lax
from jax.experimental import pallas as pl
from jax.experimental.pallas import tpu as pltpu
from jax.experimental.pallas import tpu_sc as plsc

NN = 10000
EE = 320000
FD = 128
NP = 10240
DUMMY = 10000
NTILES = 32
EPT = 10240
EP = EPT * NTILES
CH = 32
NCHUNK = EPT // CH
PADE = 128
KKS = (8000, 6400, 5120)
ROWS_B = 1024
GRID_R = NP // ROWS_B
NEGINF = float("-inf")

_HIGH = lax.Precision.HIGHEST


def _dot(a, b):
    return lax.dot_general(a, b, (((1,), (0,)), ((), ())),
                           preferred_element_type=jnp.float32,
                           precision=_HIGH)


def _dott(a, b):
    return lax.dot_general(a, b, (((0,), (0,)), ((), ())),
                           preferred_element_type=jnp.float32,
                           precision=_HIGH)


def _sortable_key(s):
    bu = lax.bitcast_convert_type(s, jnp.uint32)
    keyu = jnp.where((bu >> 31) != 0, ~bu, bu | jnp.uint32(0x80000000))
    return lax.bitcast_convert_type(keyu ^ jnp.uint32(0x80000000), jnp.int32)


def _mm_relu_body(x_ref, w_ref, b_ref, o_ref):
    o_ref[...] = jnp.maximum(_dot(x_ref[...], w_ref[...]) + b_ref[...], 0.0)


def _mm_relu(x, w, b2d):
    h = w.shape[1]
    return pl.pallas_call(
        _mm_relu_body,
        grid=(GRID_R,),
        in_specs=[pl.BlockSpec((ROWS_B, FD), lambda i: (i, 0)),
                  pl.BlockSpec((FD, h), lambda i: (0, 0)),
                  pl.BlockSpec((1, h), lambda i: (0, 0))],
        out_specs=pl.BlockSpec((ROWS_B, h), lambda i: (i, 0)),
        out_shape=jax.ShapeDtypeStruct((NP, h), jnp.float32),
    )(x, w, b2d)


def _deg_finish_body(degp_ref, h_ref, dinv_ref, curs_ref):
    ones = jnp.ones((NTILES, FD), jnp.float32)
    deg = _dott(degp_ref[...], ones) + 1.0
    dinv = lax.rsqrt(deg)
    dinv_ref[...] = dinv
    curs_ref[...] = dinv * h_ref[...]


def _deg_finish(degp, h):
    return pl.pallas_call(
        _deg_finish_body,
        grid=(GRID_R,),
        in_specs=[pl.BlockSpec((NTILES, ROWS_B), lambda i: (0, i)),
                  pl.BlockSpec((ROWS_B, FD), lambda i: (i, 0))],
        out_specs=[pl.BlockSpec((ROWS_B, FD), lambda i: (i, 0)),
                   pl.BlockSpec((ROWS_B, FD), lambda i: (i, 0))],
        out_shape=[jax.ShapeDtypeStruct((NP, FD), jnp.float32),
                   jax.ShapeDtypeStruct((NP, FD), jnp.float32)],
    )(degp, h)


def _combine1_body(part_ref, h0_ref, dinv_ref, g_ref, acc_ref, cur1_ref,
                   curs2_ref):
    agg = part_ref[0] + part_ref[1]
    dinv = dinv_ref[...]
    h0 = h0_ref[...]
    cur1 = dinv * agg + dinv * dinv * h0
    acc_ref[...] = g_ref[0] * h0 + g_ref[1] * cur1
    cur1_ref[...] = cur1
    curs2_ref[...] = dinv * cur1


def _combine1(part, h0, dinvb, g):
    return pl.pallas_call(
        _combine1_body,
        grid=(GRID_R,),
        in_specs=[pl.BlockSpec((2, ROWS_B, FD), lambda i: (0, i, 0)),
                  pl.BlockSpec((ROWS_B, FD), lambda i: (i, 0)),
                  pl.BlockSpec((ROWS_B, FD), lambda i: (i, 0)),
                  pl.BlockSpec(memory_space=pltpu.SMEM)],
        out_specs=[pl.BlockSpec((ROWS_B, FD), lambda i: (i, 0)),
                   pl.BlockSpec((ROWS_B, FD), lambda i: (i, 0)),
                   pl.BlockSpec((ROWS_B, FD), lambda i: (i, 0))],
        out_shape=[jax.ShapeDtypeStruct((NP, FD), jnp.float32)] * 3,
    )(part, h0, dinvb, g)


def _combine2_body(part_ref, cur1_ref, acc_ref, dinv_ref, g_ref, p_ref,
                   live_ref, hp_ref, sc_ref):
    agg = part_ref[0] + part_ref[1]
    dinv = dinv_ref[...]
    cur1 = cur1_ref[...]
    cur2 = dinv * agg + dinv * dinv * cur1
    hp = acc_ref[...] + g_ref[2] * cur2
    hp_ref[...] = hp
    p = p_ref[...]
    rn = lax.rsqrt(jnp.sum(p * p))
    s = _dot(hp, p) * rn
    sc_ref[...] = jnp.where(live_ref[...] > 0.0, s, NEGINF)


def _combine2(part, cur1, acc, dinvb, g, p2d, live2d):
    return pl.pallas_call(
        _combine2_body,
        grid=(GRID_R,),
        in_specs=[pl.BlockSpec((2, ROWS_B, FD), lambda i: (0, i, 0)),
                  pl.BlockSpec((ROWS_B, FD), lambda i: (i, 0)),
                  pl.BlockSpec((ROWS_B, FD), lambda i: (i, 0)),
                  pl.BlockSpec((ROWS_B, FD), lambda i: (i, 0)),
                  pl.BlockSpec(memory_space=pltpu.SMEM),
                  pl.BlockSpec((FD, 1), lambda i: (0, 0)),
                  pl.BlockSpec((ROWS_B, 1), lambda i: (i, 0))],
        out_specs=[pl.BlockSpec((ROWS_B, FD), lambda i: (i, 0)),
                   pl.BlockSpec((ROWS_B, 1), lambda i: (i, 0))],
        out_shape=[jax.ShapeDtypeStruct((NP, FD), jnp.float32),
                   jax.ShapeDtypeStruct((NP, 1), jnp.float32)],
    )(part, cur1, acc, dinvb, g, p2d, live2d)


def _topk_body(sc_ref, thr_ref, *, kk):
    key = _sortable_key(sc_ref[...])

    def vbody(i, t):
        cand = t + (jnp.int32(1) << (30 - i))
        cnt = jnp.sum((key >= cand).astype(jnp.int32))
        return jnp.where(cnt >= kk, cand, t)

    t0 = jnp.where(jnp.sum((key >= 0).astype(jnp.int32)) >= kk,
                   jnp.int32(0), jnp.int32(-2147483648))
    tval = lax.fori_loop(0, 31, vbody, t0)
    cgt = jnp.sum((key > tval).astype(jnp.int32))
    r = kk - cgt
    tie = key == tval
    ridx = (lax.broadcasted_iota(jnp.int32, key.shape, 0) * 128
            + lax.broadcasted_iota(jnp.int32, key.shape, 1))

    def ibody(i, c):
        cand = c + (jnp.int32(1) << (13 - i))
        cnt = jnp.sum((tie & (ridx < cand)).astype(jnp.int32))
        return jnp.where(cnt <= r, cand, c)

    cval = lax.fori_loop(0, 14, ibody, jnp.int32(0))
    thr_ref[...] = jnp.concatenate(
        [tval.reshape(1, 1), cval.reshape(1, 1)], axis=1)


def _topk(score2d, kk):
    return pl.pallas_call(
        functools.partial(_topk_body, kk=kk),
        in_specs=[pl.BlockSpec((NP // 128, 128), lambda: (0, 0))],
        out_specs=pl.BlockSpec((1, 2), lambda: (0, 0)),
        out_shape=jax.ShapeDtypeStruct((1, 2), jnp.int32),
    )(score2d)


def _pool_body(sc_ref, thr_ref, hp_ref, hn_ref, nl_ref, mx_ref, sm_ref, *,
               kk):
    i = pl.program_id(0)
    s = sc_ref[...]
    key = _sortable_key(s)
    tval = thr_ref[0, 0]
    cval = thr_ref[0, 1]
    ridx = lax.broadcasted_iota(jnp.int32, (ROWS_B, 1), 0) + i * ROWS_B
    keep = (key > tval) | ((key == tval) & (ridx < cval))
    keepf = keep.astype(jnp.float32)
    ones_r = jnp.ones((1, FD), jnp.float32)
    keepb = _dot(keepf, ones_r)
    th = jnp.tanh(jnp.clip(s, -30.0, 30.0))
    thb = _dot(th, ones_r)
    hn = jnp.where(keepb > 0.5, hp_ref[...] * thb, 0.0)
    hn_ref[...] = hn
    nl_ref[...] = keepf
    bmx = jnp.max(jnp.where(keepb > 0.5, hn, NEGINF), axis=0, keepdims=True)
    bsm = jnp.sum(hn, axis=0, keepdims=True) * (1.0 / kk)

    @pl.when(i == 0)
    def _():
        mx_ref[...] = jnp.full((1, FD), NEGINF, jnp.float32)
        sm_ref[...] = jnp.zeros((1, FD), jnp.float32)

    mx_ref[...] = jnp.maximum(mx_ref[...], bmx)
    sm_ref[...] = sm_ref[...] + bsm


def _pool(score, thr, hp, kk):
    return pl.pallas_call(
        functools.partial(_pool_body, kk=kk),
        grid=(GRID_R,),
        in_specs=[pl.BlockSpec((ROWS_B, 1), lambda i: (i, 0)),
                  pl.BlockSpec(memory_space=pltpu.SMEM),
                  pl.BlockSpec((ROWS_B, FD), lambda i: (i, 0))],
        out_specs=[pl.BlockSpec((ROWS_B, FD), lambda i: (i, 0)),
                   pl.BlockSpec((ROWS_B, 1), lambda i: (i, 0)),
                   pl.BlockSpec((1, FD), lambda i: (0, 0)),
                   pl.BlockSpec((1, FD), lambda i: (0, 0))],
        out_shape=[jax.ShapeDtypeStruct((NP, FD), jnp.float32),
                   jax.ShapeDtypeStruct((NP, 1), jnp.float32),
                   jax.ShapeDtypeStruct((1, FD), jnp.float32),
                   jax.ShapeDtypeStruct((1, FD), jnp.float32)],
    )(score, thr, hp)


def _head_body(xr_ref, w1_ref, b1_ref, w2_ref, b2_ref, w3_ref, b3_ref,
               o_ref):
    xs = xr_ref[0:1] + xr_ref[1:2] + xr_ref[2:3]
    o = jnp.maximum(_dot(xs, w1_ref[...]) + b1_ref[...], 0.0)
    o = jnp.maximum(_dot(o, w2_ref[...]) + b2_ref[...], 0.0)
    o = _dot(o, w3_ref[...]) + b3_ref[...]
    m = jnp.max(o, axis=1, keepdims=True)
    lse = jnp.log(jnp.sum(jnp.exp(o - m), axis=1, keepdims=True)) + m
    o_ref[...] = o - lse


def _head(xr, w1, b1, w2, b2, w3, b3):
    specs = [pl.BlockSpec(a.shape, lambda: (0,) * a.ndim)
             for a in (xr, w1, b1, w2, b2, w3, b3)]
    return pl.pallas_call(
        _head_body,
        in_specs=specs,
        out_specs=pl.BlockSpec((1, 40), lambda: (0, 0)),
        out_shape=jax.ShapeDtypeStruct((1, 40), jnp.float32),
    )(xr, w1, b1, w2, b2, w3, b3)


def _sc_edge_prep_body(src_hbm, dst_hbm, live_hbm, s2_hbm, d2_hbm, degp_hbm,
                       cnt_hbm, live_v, deg_v, src_v, dst_v, s2_v, d2_v,
                       cnt_v):
    c = lax.axis_index("c")
    s = lax.axis_index("s")
    wid = s * 2 + c
    pltpu.sync_copy(live_hbm, live_v)
    pltpu.sync_copy(src_hbm.at[wid], src_v)
    pltpu.sync_copy(dst_hbm.at[wid], dst_v)

    def zbody(i, carry):
        deg_v[pl.ds(i * 16, 16)] = jnp.zeros((16,), jnp.float32)
        return carry

    lax.fori_loop(0, NP // 16, zbody, 0)

    def vec(j, off):
        sl = pl.ds(j * 16, 16)
        sv = src_v[sl]
        dv = dst_v[sl]
        ew = plsc.load_gather(live_v, [sv]) * plsc.load_gather(live_v, [dv])
        keep = ew > 0.0
        plsc.addupdate_scatter(deg_v, [dv], ew)
        plsc.store_compressed(s2_v.at[pl.ds(off, 16)], sv, mask=keep)
        plsc.store_compressed(d2_v.at[pl.ds(off, 16)], dv, mask=keep)
        pop = plsc.all_reduce_population_count(keep)
        return off + jnp.max(pop)

    cnt = lax.fori_loop(0, EPT // 16, vec, jnp.int32(0))
    for j in range(PADE // 16):
        slj = pl.ds(cnt + j * 16, 16)
        s2_v[slj] = jnp.full((16,), DUMMY, jnp.int32)
        d2_v[slj] = jnp.full((16,), DUMMY, jnp.int32)
    cntp = ((cnt + PADE - 1) // PADE) * PADE
    cnt_v[...] = jnp.broadcast_to(cntp, (16,))
    pltpu.sync_copy(s2_v.at[pl.ds(0, EPT)], s2_hbm.at[wid])
    pltpu.sync_copy(d2_v.at[pl.ds(0, EPT)], d2_hbm.at[wid])
    pltpu.sync_copy(deg_v, degp_hbm.at[wid])
    pltpu.sync_copy(cnt_v, cnt_hbm.at[wid])


def _sc_hop_body(curs_hbm, s2_hbm, d2_hbm, cnt_hbm, part_hbm, acc, idxs_v,
                 idxd_v, rows0, rows1, rows2, rows3, cnt_v, sg0, sg1, sg2,
                 sg3, ss0, ss1, ss2, ss3):
    c = lax.axis_index("c")
    s = lax.axis_index("s")
    wid = s * 2 + c
    rpt = NP // 16
    pltpu.async_copy(s2_hbm.at[wid], idxs_v, sg1)
    pltpu.async_copy(d2_hbm.at[wid], idxd_v, sg2)
    pltpu.async_copy(cnt_hbm.at[wid], cnt_v, sg3)

    def zbody(i, carry):
        rows0[i // 8, pl.ds((i % 8) * 16, 16)] = jnp.zeros((16,), jnp.float32)
        return carry

    lax.fori_loop(0, CH * 8, zbody, 0)

    def zcopy(k, carry):
        pltpu.async_copy(rows0, acc.at[pl.ds(s * rpt + k * CH, CH)], sg0)
        return carry

    lax.fori_loop(0, rpt // CH, zcopy, 0)

    def zwait(k, carry):
        pltpu.make_async_copy(rows0, acc.at[pl.ds(s * rpt + k * CH, CH)],
                              sg0).wait()
        return carry

    lax.fori_loop(0, rpt // CH, zwait, 0)
    pltpu.make_async_copy(s2_hbm.at[wid], idxs_v, sg1).wait()
    pltpu.make_async_copy(d2_hbm.at[wid], idxd_v, sg2).wait()
    pltpu.make_async_copy(cnt_hbm.at[wid], cnt_v, sg3).wait()
    plsc.subcore_barrier()

    def gstart(ci, rows, sem):
        pltpu.async_copy(curs_hbm.at[idxs_v.at[pl.ds(ci * CH, CH)]], rows,
                         sem)

    def gwait(ci, rows, sem):
        pltpu.make_async_copy(curs_hbm.at[idxs_v.at[pl.ds(ci * CH, CH)]],
                              rows, sem).wait()

    def sstart(ci, rows, sem):
        pltpu.async_copy(rows, acc.at[idxd_v.at[pl.ds(ci * CH, CH)]], sem,
                         add=True)

    def swait(ci, rows, sem):
        pltpu.make_async_copy(rows, acc.at[idxd_v.at[pl.ds(ci * CH, CH)]],
                              sem).wait()

    nq = cnt_v[...][0] // (4 * CH)

    @pl.when(nq > 0)
    def _():
        gstart(0, rows0, sg0)
        gstart(1, rows1, sg1)

    def quad(q, carry):
        c0 = 4 * q
        gwait(c0, rows0, sg0)
        sstart(c0, rows0, ss0)

        @pl.when(q > 0)
        def _():
            swait(c0 - 2, rows2, ss2)

        gstart(c0 + 2, rows2, sg2)
        gwait(c0 + 1, rows1, sg1)
        sstart(c0 + 1, rows1, ss1)

        @pl.when(q > 0)
        def _():
            swait(c0 - 1, rows3, ss3)

        gstart(c0 + 3, rows3, sg3)
        gwait(c0 + 2, rows2, sg2)
        sstart(c0 + 2, rows2, ss2)
        swait(c0, rows0, ss0)

        @pl.when(q < nq - 1)
        def _():
            gstart(c0 + 4, rows0, sg0)

        gwait(c0 + 3, rows3, sg3)
        sstart(c0 + 3, rows3, ss3)
        swait(c0 + 1, rows1, ss1)

        @pl.when(q < nq - 1)
        def _():
            gstart(c0 + 5, rows1, sg1)

        return carry

    lax.fori_loop(0, nq, quad, 0)

    @pl.when(nq > 0)
    def _():
        swait(4 * nq - 2, rows2, ss2)
        swait(4 * nq - 1, rows3, ss3)
    plsc.subcore_barrier()
    pltpu.sync_copy(acc.at[pl.ds(s * rpt, rpt)],
                    part_hbm.at[c, pl.ds(s * rpt, rpt)])


@functools.lru_cache(maxsize=None)
def _sc_kernels():
    mesh = plsc.VectorSubcoreMesh(core_axis_name="c", subcore_axis_name="s",
                                  num_cores=2, num_subcores=16)
    params = pltpu.CompilerParams(needs_layout_passes=False)
    edge_prep = pl.kernel(
        _sc_edge_prep_body,
        out_type=(jax.ShapeDtypeStruct((NTILES, EPT), jnp.int32),
                  jax.ShapeDtypeStruct((NTILES, EPT), jnp.int32),
                  jax.ShapeDtypeStruct((NTILES, NP), jnp.float32),
                  jax.ShapeDtypeStruct((NTILES, 16), jnp.int32)),
        mesh=mesh,
        scratch_types=[pltpu.VMEM((NP,), jnp.float32),
                       pltpu.VMEM((NP,), jnp.float32),
                       pltpu.VMEM((EPT,), jnp.int32),
                       pltpu.VMEM((EPT,), jnp.int32),
                       pltpu.VMEM((EPT + PADE,), jnp.int32),
                       pltpu.VMEM((EPT + PADE,), jnp.int32),
                       pltpu.VMEM((16,), jnp.int32)],
        compiler_params=params,
    )
    hop = pl.kernel(
        _sc_hop_body,
        out_type=jax.ShapeDtypeStruct((2, NP, FD), jnp.float32),
        mesh=mesh,
        scratch_types=[pltpu.VMEM_SHARED((NP, FD), jnp.float32),
                       pltpu.VMEM((EPT,), jnp.int32),
                       pltpu.VMEM((EPT,), jnp.int32),
                       pltpu.VMEM((CH, FD), jnp.float32),
                       pltpu.VMEM((CH, FD), jnp.float32),
                       pltpu.VMEM((CH, FD), jnp.float32),
                       pltpu.VMEM((CH, FD), jnp.float32),
                       pltpu.VMEM((16,), jnp.int32),
                       pltpu.SemaphoreType.DMA,
                       pltpu.SemaphoreType.DMA,
                       pltpu.SemaphoreType.DMA,
                       pltpu.SemaphoreType.DMA,
                       pltpu.SemaphoreType.DMA,
                       pltpu.SemaphoreType.DMA,
                       pltpu.SemaphoreType.DMA,
                       pltpu.SemaphoreType.DMA],
        compiler_params=params,
    )
    return edge_prep, hop


def _sc_edge_prep(srcp, dstp, live):
    return _sc_kernels()[0](srcp, dstp, live)


def _sc_hop(curs, s2, d2, cnts):
    return _sc_kernels()[1](curs, s2, d2, cnts)


def kernel(x, edge_index, batch, W12, b12, W22, b22, W32, b32, W1, b1, W2,
           b2, W3, b3, g1, g2, g3, p1, p2, p3):
    src = edge_index[0]
    dst = edge_index[1]
    pad_e = jnp.full((EP - EE,), DUMMY, jnp.int32)
    srcp = jnp.concatenate([src, pad_e]).reshape(NTILES, EPT)
    dstp = jnp.concatenate([dst, pad_e]).reshape(NTILES, EPT)
    xp = jnp.pad(x, ((0, NP - NN), (0, 0)))
    live = (jnp.arange(NP) < NN).astype(jnp.float32)

    h = _mm_relu(xp, W12, b12.reshape(1, -1))
    gs = (g1, g2, g3)
    ps = (p1, p2, p3)
    nxt = ((W22, b22), (W32, b32), None)
    xrs = []
    for l in range(3):
        s2, d2, degp, cnts = _sc_edge_prep(srcp, dstp, live)
        dinvb, curs = _deg_finish(degp, h)
        part = _sc_hop(curs, s2, d2, cnts)
        accv, cur1, curs2 = _combine1(part, h, dinvb, gs[l])
        part2 = _sc_hop(curs2, s2, d2, cnts)
        hp, score = _combine2(part2, cur1, accv, dinvb, gs[l],
                              ps[l].reshape(-1, 1), live.reshape(-1, 1))
        thr = _topk(score.reshape(NP // 128, 128), KKS[l])
        hn, nl, mx, sm = _pool(score, thr, hp, KKS[l])
        xrs.append(jnp.concatenate([mx, sm], axis=1))
        live = nl.reshape(-1)
        if nxt[l] is not None:
            h = _mm_relu(hn, nxt[l][0], nxt[l][1].reshape(1, -1))
    return _head(jnp.concatenate(xrs, 0), W1, b1.reshape(1, -1),
                 W2, b2.reshape(1, -1), W3, b3.reshape(1, -1))
